# Initial kernel scaffold; baseline (speedup 1.0000x reference)
import jax
import jax.numpy as jnp
from jax.experimental import pallas as pl


def kernel(x, W_r, b_r, W_e, b_e):
    raise NotImplementedError("write your pallas kernel here")


# speedup vs baseline: 1.7040x; 1.7040x over previous
import functools

import jax
import jax.numpy as jnp
from jax import lax
from jax.experimental import pallas as pl
from jax.experimental.pallas import tpu as pltpu
from jax.experimental.pallas import tpu_sc as plsc

_LANES = 16


def _router_body(xf_ref, wr_ref, br_ref, s_ref):
    logits = jnp.dot(xf_ref[...], wr_ref[...],
                     preferred_element_type=jnp.float32) + br_ref[...]
    m = jnp.max(logits, axis=-1, keepdims=True)
    ex = jnp.exp(logits - m)
    s_ref[...] = ex / jnp.sum(ex, axis=-1, keepdims=True)


def _make_sc_topk(E, bs, k):
    nvec = bs // _LANES
    mesh = plsc.VectorSubcoreMesh(core_axis_name="c", subcore_axis_name="s")

    def body(s_hbm, w_hbm, s_v, w_v):
        cid = lax.axis_index("c")
        sid = lax.axis_index("s")
        wid = sid * 2 + cid

        @pl.when(wid < E)
        def _():
            pltpu.sync_copy(s_hbm.at[wid], s_v)
            zero = jnp.zeros((_LANES,), jnp.int32)
            kvec = jnp.full((_LANES,), k, jnp.int32)
            one = jnp.full((_LANES,), 1, jnp.int32)

            def count(pred_thr, strict):
                def inner(i, cnt):
                    v = s_v[pl.ds(i * _LANES, _LANES)]
                    m = (v > pred_thr) if strict else (v >= pred_thr)
                    return cnt + plsc.all_reduce_population_count(m)
                return lax.fori_loop(0, nvec, inner, zero)

            def outer(j, thr):
                cand = thr | (one << (30 - j))
                return jnp.where(count(cand, False) >= kvec, cand, thr)

            thr = lax.fori_loop(0, 31, outer, zero)
            r = kvec - count(thr, True)

            def emit(i, run):
                v = s_v[pl.ds(i * _LANES, _LANES)]
                gt = v > thr
                eq = v == thr
                eqc = plsc.cumsum(eq.astype(jnp.int32))
                inc = gt | (eq & ((run + eqc) <= r))
                w_v[pl.ds(i * _LANES, _LANES)] = jnp.where(inc, v, zero)
                return run + plsc.all_reduce_population_count(eq)

            lax.fori_loop(0, nvec, emit, zero)
            pltpu.sync_copy(w_v, w_hbm.at[wid])

    return pl.kernel(
        body,
        out_type=jax.ShapeDtypeStruct((E, bs), jnp.int32),
        mesh=mesh,
        compiler_params=pltpu.CompilerParams(needs_layout_passes=False),
        scratch_types=[
            pltpu.VMEM((bs,), jnp.int32),
            pltpu.VMEM((bs,), jnp.int32),
        ],
    )


def _ffn_body(xf_ref, we_ref, be_ref, w8t_ref, y_ref):
    w = jnp.sum(w8t_ref[...], axis=1, keepdims=True)
    y0 = jnp.dot(xf_ref[...], we_ref[...],
                 preferred_element_type=jnp.float32) + be_ref[...]
    y_ref[...] = y0 * w


def kernel(x, W_r, b_r, W_e, b_e):
    b, s, h = x.shape
    bs = b * s
    E = W_r.shape[1]
    k = min(int(bs * 1.0 / E), bs)
    xf = x.reshape(bs, h)

    S = pl.pallas_call(
        _router_body,
        out_shape=jax.ShapeDtypeStruct((bs, E), jnp.float32),
    )(xf, W_r, b_r.reshape(1, E))

    s_bits = lax.bitcast_convert_type(S.T, jnp.int32)
    w8_bits = _make_sc_topk(E, bs, k)(s_bits)
    w8 = lax.bitcast_convert_type(w8_bits, jnp.float32)

    m_blk = 256
    grid = (bs // m_blk,)
    y = pl.pallas_call(
        _ffn_body,
        grid=grid,
        in_specs=[
            pl.BlockSpec((m_blk, h), lambda i: (i, 0)),
            pl.BlockSpec((h, h), lambda i: (0, 0)),
            pl.BlockSpec((1, h), lambda i: (0, 0)),
            pl.BlockSpec((m_blk, E), lambda i: (i, 0)),
        ],
        out_specs=pl.BlockSpec((m_blk, h), lambda i: (i, 0)),
        out_shape=jax.ShapeDtypeStruct((bs, h), jnp.float32),
    )(xf, W_e, b_e.reshape(1, h), w8.T)

    return y.reshape(b, s, h)


# speedup vs baseline: 2.1657x; 1.2709x over previous
import functools

import jax
import jax.numpy as jnp
from jax import lax
from jax.experimental import pallas as pl
from jax.experimental.pallas import tpu as pltpu
from jax.experimental.pallas import tpu_sc as plsc

_LANES = 16


def _router_body(xf_ref, wr_ref, br_ref, s_ref):
    logits = jnp.dot(xf_ref[...], wr_ref[...],
                     preferred_element_type=jnp.float32) + br_ref[...]
    m = jnp.max(logits, axis=-1, keepdims=True)
    ex = jnp.exp(logits - m)
    s_ref[...] = ex / jnp.sum(ex, axis=-1, keepdims=True)


def _make_sc_topk(E, bs, k):
    nvec = bs // _LANES
    mesh = plsc.VectorSubcoreMesh(core_axis_name="c", subcore_axis_name="s")

    def body(s_hbm, w_hbm, s_v, w_v, hist_v):
        cid = lax.axis_index("c")
        sid = lax.axis_index("s")
        wid = sid * 2 + cid

        @pl.when(wid < E)
        def _():
            pltpu.sync_copy(s_hbm.at[wid], s_v)
            zero = jnp.zeros((_LANES,), jnp.int32)
            onev = jnp.full((_LANES,), 1, jnp.int32)
            lane = lax.iota(jnp.int32, _LANES)

            def digit_pass(carry, shift, nbits):
                prefix, kk = carry
                hi = shift + nbits

                def clear(i, _):
                    hist_v[pl.ds(i * _LANES, _LANES)] = zero
                    return 0
                lax.fori_loop(0, 256 // _LANES, clear, 0)

                def accum(i, _):
                    v = s_v[pl.ds(i * _LANES, _LANES)]
                    dig = (v >> shift) & 255
                    inc = lax.shift_right_logical(v, hi) == prefix
                    plsc.addupdate_scatter(hist_v, [dig], onev, mask=inc)
                    return 0
                lax.fori_loop(0, nvec, accum, 0)

                def scan(j, c):
                    total, found, dstar, s_above = c
                    vh = (256 // _LANES) - 1 - j
                    h = hist_v[pl.ds(vh * _LANES, _LANES)]
                    hr = lax.rev(h, (0,))
                    rc = plsc.cumsum(hr)
                    sfx = total + rc
                    m = sfx >= kk
                    anym = jnp.max(plsc.all_reduce_population_count(m)) > 0
                    ffs = jnp.max(plsc.all_reduce_ffs(m))
                    dcand = vh * _LANES + (_LANES - 1) - ffs
                    sa_v = jnp.where(lane == ffs, sfx - hr, zero)
                    sacand = jnp.max(sa_v)
                    upd = jnp.logical_and(jnp.logical_not(found), anym)
                    dstar = jnp.where(upd, dcand, dstar)
                    s_above = jnp.where(upd, sacand, s_above)
                    return (total + jnp.max(rc),
                            jnp.logical_or(found, anym), dstar, s_above)

                _, _, dstar, s_above = lax.fori_loop(
                    0, 256 // _LANES, scan,
                    (jnp.int32(0), jnp.bool_(False), jnp.int32(0),
                     jnp.int32(0)))
                return ((prefix << nbits) | dstar, kk - s_above)

            carry = (jnp.int32(0), jnp.int32(k))
            for shift, nbits in ((23, 8), (15, 8), (7, 8), (0, 7)):
                carry = digit_pass(carry, shift, nbits)
            thr, r = carry

            def emit(i, run):
                v = s_v[pl.ds(i * _LANES, _LANES)]
                gt = v > thr
                eq = v == thr
                eqc = plsc.cumsum(eq.astype(jnp.int32))
                inc = gt | (eq & ((run + eqc) <= r))
                w_v[pl.ds(i * _LANES, _LANES)] = jnp.where(inc, v, zero)
                return run + jnp.max(plsc.all_reduce_population_count(eq))

            lax.fori_loop(0, nvec, emit, jnp.int32(0))
            pltpu.sync_copy(w_v, w_hbm.at[wid])

    return pl.kernel(
        body,
        out_type=jax.ShapeDtypeStruct((E, bs), jnp.int32),
        mesh=mesh,
        compiler_params=pltpu.CompilerParams(needs_layout_passes=False),
        scratch_types=[
            pltpu.VMEM((bs,), jnp.int32),
            pltpu.VMEM((bs,), jnp.int32),
            pltpu.VMEM((256,), jnp.int32),
        ],
    )


def _ffn_body(xf_ref, we_ref, be_ref, w8t_ref, y_ref):
    w = jnp.sum(w8t_ref[...], axis=1, keepdims=True)
    y0 = jnp.dot(xf_ref[...], we_ref[...],
                 preferred_element_type=jnp.float32) + be_ref[...]
    y_ref[...] = y0 * w


def kernel(x, W_r, b_r, W_e, b_e):
    b, s, h = x.shape
    bs = b * s
    E = W_r.shape[1]
    k = min(int(bs * 1.0 / E), bs)
    xf = x.reshape(bs, h)

    S = pl.pallas_call(
        _router_body,
        out_shape=jax.ShapeDtypeStruct((bs, E), jnp.float32),
    )(xf, W_r, b_r.reshape(1, E))

    s_bits = lax.bitcast_convert_type(S.T, jnp.int32)
    w8_bits = _make_sc_topk(E, bs, k)(s_bits)
    w8 = lax.bitcast_convert_type(w8_bits, jnp.float32)

    m_blk = 256
    grid = (bs // m_blk,)
    y = pl.pallas_call(
        _ffn_body,
        grid=grid,
        in_specs=[
            pl.BlockSpec((m_blk, h), lambda i: (i, 0)),
            pl.BlockSpec((h, h), lambda i: (0, 0)),
            pl.BlockSpec((1, h), lambda i: (0, 0)),
            pl.BlockSpec((m_blk, E), lambda i: (i, 0)),
        ],
        out_specs=pl.BlockSpec((m_blk, h), lambda i: (i, 0)),
        out_shape=jax.ShapeDtypeStruct((bs, h), jnp.float32),
    )(xf, W_e, b_e.reshape(1, h), w8.T)

    return y.reshape(b, s, h)


# speedup vs baseline: 2.1674x; 1.0008x over previous
import functools

import jax
import jax.numpy as jnp
from jax import lax
from jax.experimental import pallas as pl
from jax.experimental.pallas import tpu as pltpu
from jax.experimental.pallas import tpu_sc as plsc

_LANES = 16


def _router_body(xf_ref, wr_ref, br_ref, s_ref):
    logits = jnp.dot(xf_ref[...], wr_ref[...],
                     preferred_element_type=jnp.float32) + br_ref[...]
    m = jnp.max(logits, axis=-1, keepdims=True)
    ex = jnp.exp(logits - m)
    s_ref[...] = ex / jnp.sum(ex, axis=-1, keepdims=True)


def _make_sc_topk(E, bs, k):
    nvec = bs // _LANES
    mesh = plsc.VectorSubcoreMesh(core_axis_name="c", subcore_axis_name="s")

    def body(s_hbm, w_hbm, s_v, w_v, hist_v):
        cid = lax.axis_index("c")
        sid = lax.axis_index("s")
        wid = sid * 2 + cid

        @pl.when(wid < E)
        def _():
            pltpu.sync_copy(s_hbm.at[wid], s_v)
            zero = jnp.zeros((_LANES,), jnp.int32)
            onev = jnp.full((_LANES,), 1, jnp.int32)
            lane = lax.iota(jnp.int32, _LANES)

            def digit_pass(carry, shift, nbits):
                prefix, kk = carry
                hi = shift + nbits

                def clear(i, _):
                    hist_v[pl.ds(i * _LANES, _LANES)] = zero
                    return 0
                lax.fori_loop(0, 256 // _LANES, clear, 0)

                def accum(i, _):
                    v = s_v[pl.ds(i * _LANES, _LANES)]
                    dig = (v >> shift) & 255
                    inc = lax.shift_right_logical(v, hi) == prefix
                    plsc.addupdate_scatter(hist_v, [dig], onev, mask=inc)
                    return 0
                lax.fori_loop(0, nvec, accum, 0)

                def scan(j, c):
                    total, found, dstar, s_above = c
                    vh = (256 // _LANES) - 1 - j
                    h = hist_v[pl.ds(vh * _LANES, _LANES)]
                    hr = lax.rev(h, (0,))
                    rc = plsc.cumsum(hr)
                    sfx = total + rc
                    m = sfx >= kk
                    anym = jnp.max(plsc.all_reduce_population_count(m)) > 0
                    ffs = jnp.max(plsc.all_reduce_ffs(m))
                    dcand = vh * _LANES + (_LANES - 1) - ffs
                    sa_v = jnp.where(lane == ffs, sfx - hr, zero)
                    sacand = jnp.max(sa_v)
                    upd = jnp.logical_and(jnp.logical_not(found), anym)
                    dstar = jnp.where(upd, dcand, dstar)
                    s_above = jnp.where(upd, sacand, s_above)
                    return (total + jnp.max(rc),
                            jnp.logical_or(found, anym), dstar, s_above)

                _, _, dstar, s_above = lax.fori_loop(
                    0, 256 // _LANES, scan,
                    (jnp.int32(0), jnp.bool_(False), jnp.int32(0),
                     jnp.int32(0)))
                return ((prefix << nbits) | dstar, kk - s_above)

            carry = (jnp.int32(0), jnp.int32(k))
            for shift, nbits in ((23, 8), (15, 8), (7, 8), (0, 7)):
                carry = digit_pass(carry, shift, nbits)
            thr, r = carry

            def emit(i, run):
                v = s_v[pl.ds(i * _LANES, _LANES)]
                gt = v > thr
                eq = v == thr
                eqc = plsc.cumsum(eq.astype(jnp.int32))
                inc = gt | (eq & ((run + eqc) <= r))
                w_v[pl.ds(i * _LANES, _LANES)] = jnp.where(inc, v, zero)
                return run + jnp.max(plsc.all_reduce_population_count(eq))

            lax.fori_loop(0, nvec, emit, jnp.int32(0))
            pltpu.sync_copy(w_v, w_hbm.at[wid])

    return pl.kernel(
        body,
        out_type=jax.ShapeDtypeStruct((E, bs), jnp.int32),
        mesh=mesh,
        compiler_params=pltpu.CompilerParams(needs_layout_passes=False),
        scratch_types=[
            pltpu.VMEM((bs,), jnp.int32),
            pltpu.VMEM((bs,), jnp.int32),
            pltpu.VMEM((256,), jnp.int32),
        ],
    )


def _ffn_body(xf_ref, we_ref, be_ref, y0_ref):
    y0_ref[...] = jnp.dot(xf_ref[...], we_ref[...],
                          preferred_element_type=jnp.float32) + be_ref[...]


def _scale_body(y0_ref, w8t_ref, y_ref):
    w = jnp.sum(w8t_ref[...], axis=1, keepdims=True)
    y_ref[...] = y0_ref[...] * w


def kernel(x, W_r, b_r, W_e, b_e):
    b, s, h = x.shape
    bs = b * s
    E = W_r.shape[1]
    k = min(int(bs * 1.0 / E), bs)
    xf = x.reshape(bs, h)

    S = pl.pallas_call(
        _router_body,
        out_shape=jax.ShapeDtypeStruct((bs, E), jnp.float32),
    )(xf, W_r, b_r.reshape(1, E))

    s_bits = lax.bitcast_convert_type(S.T, jnp.int32)
    w8_bits = _make_sc_topk(E, bs, k)(s_bits)
    w8 = lax.bitcast_convert_type(w8_bits, jnp.float32)

    m_blk = 256
    grid = (bs // m_blk,)
    y0 = pl.pallas_call(
        _ffn_body,
        grid=grid,
        in_specs=[
            pl.BlockSpec((m_blk, h), lambda i: (i, 0)),
            pl.BlockSpec((h, h), lambda i: (0, 0)),
            pl.BlockSpec((1, h), lambda i: (0, 0)),
        ],
        out_specs=pl.BlockSpec((m_blk, h), lambda i: (i, 0)),
        out_shape=jax.ShapeDtypeStruct((bs, h), jnp.float32),
    )(xf, W_e, b_e.reshape(1, h))

    y = pl.pallas_call(
        _scale_body,
        grid=grid,
        in_specs=[
            pl.BlockSpec((m_blk, h), lambda i: (i, 0)),
            pl.BlockSpec((m_blk, E), lambda i: (i, 0)),
        ],
        out_specs=pl.BlockSpec((m_blk, h), lambda i: (i, 0)),
        out_shape=jax.ShapeDtypeStruct((bs, h), jnp.float32),
    )(y0, w8.T)

    return y.reshape(b, s, h)
